# Initial kernel scaffold; baseline (speedup 1.0000x reference)
import jax
import jax.numpy as jnp
from jax.experimental import pallas as pl


def kernel(features, modality_ids, mode_ids, W0, b0, W1, b1, W2, b2, mode_table, Ws, bs, Wh, bh):
    raise NotImplementedError("write your pallas kernel here")


# speedup vs baseline: 1.2863x; 1.2863x over previous
import functools

import jax
import jax.numpy as jnp
from jax.experimental import pallas as pl

B = 4096
D0 = 2048
D1 = 1024
D2 = 512
LATENT = 2048
NUM_MODES = 64
MODE_DIM = 512

TB = 256
JB = 512


def _tables_body(mt_ref, ws_ref, bs_ref, wh_ref, bh_ref, st_ref, ht_ref):
    mt = mt_ref[...]
    dn = (((1,), (1,)), ((), ()))
    st_ref[...] = jax.lax.dot_general(
        mt, ws_ref[...], dn, preferred_element_type=jnp.float32) + bs_ref[...]
    ht_ref[...] = jax.lax.dot_general(
        mt, wh_ref[...], dn, preferred_element_type=jnp.float32) + bh_ref[...]


def _film_tables(mode_table, Ws, bs, Wh, bh):
    return pl.pallas_call(
        _tables_body,
        out_shape=(
            jax.ShapeDtypeStruct((NUM_MODES, LATENT), jnp.float32),
            jax.ShapeDtypeStruct((NUM_MODES, LATENT), jnp.float32),
        ),
    )(mode_table, Ws, bs.reshape(1, LATENT), Wh, bh.reshape(1, LATENT))


def _main_body(x_ref, mod_ref, mode_ref, w0_ref, b0_ref, w1_ref, b1_ref,
               w2_ref, b2_ref, st_ref, ht_ref, out_ref):
    x = x_ref[...]
    dn = (((1,), (1,)), ((), ()))
    p0 = jax.lax.dot_general(x, w0_ref[...], dn,
                             preferred_element_type=jnp.float32) + b0_ref[...]
    p1 = jax.lax.dot_general(x[:, :D1], w1_ref[...], dn,
                             preferred_element_type=jnp.float32) + b1_ref[...]
    p2 = jax.lax.dot_general(x[:, :D2], w2_ref[...], dn,
                             preferred_element_type=jnp.float32) + b2_ref[...]
    mod = mod_ref[0, 0, :]
    modc = mod[:, None]
    content = jnp.where(modc == 0, p0, jnp.where(modc == 1, p1, p2))

    mode = mode_ref[0, 0, :]
    iota = jax.lax.broadcasted_iota(jnp.int32, (TB, NUM_MODES), 1)
    onehot = (mode[:, None] == iota).astype(jnp.bfloat16)
    s = jax.lax.dot_general(onehot, st_ref[...], (((1,), (0,)), ((), ())),
                            preferred_element_type=jnp.float32)
    h = jax.lax.dot_general(onehot, ht_ref[...], (((1,), (0,)), ((), ())),
                            preferred_element_type=jnp.float32)
    out_ref[...] = content * (1.0 + s) + h


@jax.jit
def kernel(features, modality_ids, mode_ids, W0, b0, W1, b1, W2, b2,
           mode_table, Ws, bs, Wh, bh):
    st, ht = _film_tables(mode_table, Ws, bs, Wh, bh)
    xb = features.astype(jnp.bfloat16)
    w0 = W0.astype(jnp.bfloat16)
    w1 = W1.astype(jnp.bfloat16)
    w2 = W2.astype(jnp.bfloat16)
    stb = st.astype(jnp.bfloat16)
    htb = ht.astype(jnp.bfloat16)
    mod3 = modality_ids.reshape(B // TB, 1, TB)
    mode3 = mode_ids.reshape(B // TB, 1, TB)

    grid = (LATENT // JB, B // TB)
    out = pl.pallas_call(
        _main_body,
        grid=grid,
        in_specs=[
            pl.BlockSpec((TB, D0), lambda j, i: (i, 0)),
            pl.BlockSpec((1, 1, TB), lambda j, i: (i, 0, 0)),
            pl.BlockSpec((1, 1, TB), lambda j, i: (i, 0, 0)),
            pl.BlockSpec((JB, D0), lambda j, i: (j, 0)),
            pl.BlockSpec((1, JB), lambda j, i: (0, j)),
            pl.BlockSpec((JB, D1), lambda j, i: (j, 0)),
            pl.BlockSpec((1, JB), lambda j, i: (0, j)),
            pl.BlockSpec((JB, D2), lambda j, i: (j, 0)),
            pl.BlockSpec((1, JB), lambda j, i: (0, j)),
            pl.BlockSpec((NUM_MODES, JB), lambda j, i: (0, j)),
            pl.BlockSpec((NUM_MODES, JB), lambda j, i: (0, j)),
        ],
        out_specs=pl.BlockSpec((TB, JB), lambda j, i: (i, j)),
        out_shape=jax.ShapeDtypeStruct((B, LATENT), jnp.float32),
    )(xb, mod3, mode3, w0, b0.reshape(1, LATENT), w1, b1.reshape(1, LATENT),
      w2, b2.reshape(1, LATENT), stb, htb)
    return out
